# baseline (speedup 1.0000x reference)
import functools

import jax
import jax.numpy as jnp
from jax import lax
from jax.experimental import pallas as pl
from jax.experimental.pallas import tpu as pltpu
from jax.experimental.pallas import tpu_sc as plsc

N = 4194304
NUM_SEGMENTS = 64
_INFO = plsc.get_sparse_core_info()
NC = _INFO.num_cores
NS = _INFO.num_subcores
NW = NC * NS
PER_W = N // NW
CH = 8192
NCH = PER_W // CH
L = 16


def _spline_body(x_hbm, cp_hbm, dv_hbm, out_hbm,
                 cp_v, dv_v, ta_v, tb_v, tc_v, td_v, xb0, xb1, yb0, yb1,
                 sem_in0, sem_in1, sem_out0, sem_out1):
    wid = lax.axis_index("s") * NC + lax.axis_index("c")
    base = wid * PER_W

    xb = (xb0, xb1)
    yb = (yb0, yb1)
    sem_in = (sem_in0, sem_in1)
    sem_out = (sem_out0, sem_out1)

    cp_in = [None, None]
    cp_in[0] = pltpu.async_copy(x_hbm.at[pl.ds(base, CH)], xb[0], sem_in[0])

    pltpu.sync_copy(cp_hbm, cp_v)
    pltpu.sync_copy(dv_hbm, dv_v)
    for j in range(NUM_SEGMENTS // L):
        off = j * L
        y0 = cp_v[pl.ds(off, L)]
        y1 = cp_v[pl.ds(off + 1, L)]
        d0 = dv_v[pl.ds(off, L)]
        d1 = dv_v[pl.ds(off + 1, L)]
        dy = y1 - y0
        ta_v[pl.ds(off, L)] = y0
        tb_v[pl.ds(off, L)] = d0 * 0.0625
        tc_v[pl.ds(off, L)] = (48.0 * dy - 2.0 * d0 - d1) * 0.0625
        td_v[pl.ds(off, L)] = (-32.0 * dy + d0 + d1) * 0.0625

    cp_in[1] = pltpu.async_copy(
        x_hbm.at[pl.ds(base + CH, CH)], xb[1], sem_in[1])

    def _compute(xref, yref):
        @plsc.parallel_loop(0, CH, step=L, unroll=16)
        def _vec_step(off):
            xv = xref[pl.ds(off, L)]
            xs = xv * 16.0 + 32.0
            xc = jnp.minimum(jnp.maximum(xs, 0.0), float(NUM_SEGMENTS - 1))
            seg = xc.astype(jnp.int32)
            u = xs - seg.astype(jnp.float32)
            ca = plsc.load_gather(ta_v, [seg])
            cb = plsc.load_gather(tb_v, [seg])
            cc = plsc.load_gather(tc_v, [seg])
            cd = plsc.load_gather(td_v, [seg])
            yref[pl.ds(off, L)] = ca + u * (cb + u * (cc + u * cd))

    def _pair(g, carry):
        for b in range(2):
            off = base + (2 * g + b) * CH
            @pl.when(g > 0)
            def _():
                pltpu.make_async_copy(
                    yb[b], out_hbm.at[pl.ds(off, CH)], sem_out[b]).wait()

            pltpu.make_async_copy(
                x_hbm.at[pl.ds(off, CH)], xb[b], sem_in[b]).wait()
            _compute(xb[b], yb[b])
            pltpu.async_copy(
                yb[b], out_hbm.at[pl.ds(off, CH)], sem_out[b])

            @pl.when(g < NCH // 2 - 1)
            def _():
                pltpu.async_copy(
                    x_hbm.at[pl.ds(off + 2 * CH, CH)], xb[b], sem_in[b])
        return carry

    lax.fori_loop(0, NCH // 2, _pair, 0)

    pltpu.make_async_copy(yb[0], out_hbm.at[pl.ds(base, CH)], sem_out[0]).wait()
    pltpu.make_async_copy(yb[1], out_hbm.at[pl.ds(base, CH)], sem_out[1]).wait()


_spline_sc = pl.kernel(
    _spline_body,
    out_type=jax.ShapeDtypeStruct((N,), jnp.float32),
    mesh=plsc.VectorSubcoreMesh(core_axis_name="c", subcore_axis_name="s"),
    scratch_types=[
        pltpu.VMEM((NUM_SEGMENTS + 1,), jnp.float32),
        pltpu.VMEM((NUM_SEGMENTS + 1,), jnp.float32),
        pltpu.VMEM((NUM_SEGMENTS,), jnp.float32),
        pltpu.VMEM((NUM_SEGMENTS,), jnp.float32),
        pltpu.VMEM((NUM_SEGMENTS,), jnp.float32),
        pltpu.VMEM((NUM_SEGMENTS,), jnp.float32),
        pltpu.VMEM((CH,), jnp.float32),
        pltpu.VMEM((CH,), jnp.float32),
        pltpu.VMEM((CH,), jnp.float32),
        pltpu.VMEM((CH,), jnp.float32),
        pltpu.SemaphoreType.DMA,
        pltpu.SemaphoreType.DMA,
        pltpu.SemaphoreType.DMA,
        pltpu.SemaphoreType.DMA,
    ],
    compiler_params=pltpu.CompilerParams(needs_layout_passes=False),
)


@jax.jit
def kernel(x, control_points, derivatives):
    y = _spline_sc(x, control_points[:, 0], derivatives[:, 0])
    return y[:, None]


# speedup vs baseline: 1.3280x; 1.3280x over previous
import functools

import jax
import jax.numpy as jnp
from jax import lax
from jax.experimental import pallas as pl
from jax.experimental.pallas import tpu as pltpu
from jax.experimental.pallas import tpu_sc as plsc

N = 4194304
NUM_SEGMENTS = 64
_INFO = plsc.get_sparse_core_info()
NC = _INFO.num_cores
NS = _INFO.num_subcores
NW = NC * NS
PER_W = N // NW
CH = 8192
NCH = PER_W // CH
L = 16


def _spline_body(x_hbm, cp_hbm, dv_hbm, out_hbm,
                 cp_v, dv_v, ta_v, tb_v, tc_v, td_v, xb0, xb1, yb0, yb1,
                 sem_in0, sem_in1, sem_out0, sem_out1):
    wid = lax.axis_index("s") * NC + lax.axis_index("c")
    base = wid * PER_W

    xb = (xb0, xb1)
    yb = (yb0, yb1)
    sem_in = (sem_in0, sem_in1)
    sem_out = (sem_out0, sem_out1)

    cp_in = [None, None]
    cp_in[0] = pltpu.async_copy(x_hbm.at[pl.ds(base, CH)], xb[0], sem_in[0])
    cp_in[1] = pltpu.async_copy(
        x_hbm.at[pl.ds(base + CH, CH)], xb[1], sem_in[1])

    pltpu.sync_copy(cp_hbm, cp_v)
    pltpu.sync_copy(dv_hbm, dv_v)
    for j in range(NUM_SEGMENTS // L):
        off = j * L
        y0 = cp_v[pl.ds(off, L)]
        y1 = cp_v[pl.ds(off + 1, L)]
        d0 = dv_v[pl.ds(off, L)]
        d1 = dv_v[pl.ds(off + 1, L)]
        dy = y1 - y0
        ta_v[pl.ds(off, L)] = y0
        tb_v[pl.ds(off, L)] = d0 * 0.0625
        tc_v[pl.ds(off, L)] = (48.0 * dy - 2.0 * d0 - d1) * 0.0625
        td_v[pl.ds(off, L)] = (-32.0 * dy + d0 + d1) * 0.0625

    def _compute(xref, yref):
        @plsc.parallel_loop(0, CH, step=L, unroll=8)
        def _vec_step(off):
            xv = xref[pl.ds(off, L)]
            xs = xv * 16.0 + 32.0
            xc = jnp.minimum(jnp.maximum(xs, 0.0), float(NUM_SEGMENTS - 1))
            seg = xc.astype(jnp.int32)
            u = xs - seg.astype(jnp.float32)
            ca = plsc.load_gather(ta_v, [seg])
            cb = plsc.load_gather(tb_v, [seg])
            cc = plsc.load_gather(tc_v, [seg])
            cd = plsc.load_gather(td_v, [seg])
            yref[pl.ds(off, L)] = ca + u * (cb + u * (cc + u * cd))

    def _pair(g, carry):
        for b in range(2):
            off = base + (2 * g + b) * CH
            @pl.when(g > 0)
            def _():
                pltpu.make_async_copy(
                    yb[b], out_hbm.at[pl.ds(off, CH)], sem_out[b]).wait()

            pltpu.make_async_copy(
                x_hbm.at[pl.ds(off, CH)], xb[b], sem_in[b]).wait()
            _compute(xb[b], yb[b])
            pltpu.async_copy(
                yb[b], out_hbm.at[pl.ds(off, CH)], sem_out[b])

            @pl.when(g < NCH // 2 - 1)
            def _():
                pltpu.async_copy(
                    x_hbm.at[pl.ds(off + 2 * CH, CH)], xb[b], sem_in[b])
        return carry

    lax.fori_loop(0, NCH // 2, _pair, 0)

    pltpu.make_async_copy(yb[0], out_hbm.at[pl.ds(base, CH)], sem_out[0]).wait()
    pltpu.make_async_copy(yb[1], out_hbm.at[pl.ds(base, CH)], sem_out[1]).wait()


_spline_sc = pl.kernel(
    _spline_body,
    out_type=jax.ShapeDtypeStruct((N,), jnp.float32),
    mesh=plsc.VectorSubcoreMesh(core_axis_name="c", subcore_axis_name="s"),
    scratch_types=[
        pltpu.VMEM((NUM_SEGMENTS + 1,), jnp.float32),
        pltpu.VMEM((NUM_SEGMENTS + 1,), jnp.float32),
        pltpu.VMEM((NUM_SEGMENTS,), jnp.float32),
        pltpu.VMEM((NUM_SEGMENTS,), jnp.float32),
        pltpu.VMEM((NUM_SEGMENTS,), jnp.float32),
        pltpu.VMEM((NUM_SEGMENTS,), jnp.float32),
        pltpu.VMEM((CH,), jnp.float32),
        pltpu.VMEM((CH,), jnp.float32),
        pltpu.VMEM((CH,), jnp.float32),
        pltpu.VMEM((CH,), jnp.float32),
        pltpu.SemaphoreType.DMA,
        pltpu.SemaphoreType.DMA,
        pltpu.SemaphoreType.DMA,
        pltpu.SemaphoreType.DMA,
    ],
    compiler_params=pltpu.CompilerParams(needs_layout_passes=False),
)


@jax.jit
def kernel(x, control_points, derivatives):
    y = _spline_sc(x, control_points[:, 0], derivatives[:, 0])
    return y[:, None]


# speedup vs baseline: 1.3319x; 1.0030x over previous
import jax
import jax.numpy as jnp
from jax import lax
from jax.experimental import pallas as pl
from jax.experimental.pallas import tpu as pltpu
from jax.experimental.pallas import tpu_sc as plsc

N = 4194304
NUM_SEGMENTS = 64
_INFO = plsc.get_sparse_core_info()
NC = _INFO.num_cores
NS = _INFO.num_subcores
NW = NC * NS
PER_W = N // NW
CH = 8192
NCH = PER_W // CH
L = 16


def _spline_body(x_hbm, cp_hbm, dv_hbm, out_hbm,
                 cp_v, dv_v, ta_v, tb_v, tc_v, td_v, xb0, xb1, yb0, yb1,
                 sem_in0, sem_in1, sem_out0, sem_out1):
    wid = lax.axis_index("s") * NC + lax.axis_index("c")
    base = wid * PER_W

    xb = (xb0, xb1)
    yb = (yb0, yb1)
    sem_in = (sem_in0, sem_in1)
    sem_out = (sem_out0, sem_out1)

    pltpu.async_copy(x_hbm.at[pl.ds(base, CH)], xb[0], sem_in[0])
    pltpu.async_copy(x_hbm.at[pl.ds(base + CH, CH)], xb[1], sem_in[1])

    pltpu.sync_copy(cp_hbm, cp_v)
    pltpu.sync_copy(dv_hbm, dv_v)
    for j in range(NUM_SEGMENTS // L):
        off = j * L
        y0 = cp_v[pl.ds(off, L)]
        y1 = cp_v[pl.ds(off + 1, L)]
        d0 = dv_v[pl.ds(off, L)]
        d1 = dv_v[pl.ds(off + 1, L)]
        dy = y1 - y0
        ta_v[pl.ds(off, L)] = y0
        tb_v[pl.ds(off, L)] = d0 * 0.0625
        tc_v[pl.ds(off, L)] = (48.0 * dy - 2.0 * d0 - d1) * 0.0625
        td_v[pl.ds(off, L)] = (-32.0 * dy + d0 + d1) * 0.0625

    def _compute(xref, yref):
        @plsc.parallel_loop(0, CH, step=L, unroll=8)
        def _vec_step(off):
            xv = xref[pl.ds(off, L)]
            xs = xv * 16.0 + 32.0
            xc = jnp.minimum(jnp.maximum(xs, 0.0), float(NUM_SEGMENTS - 1))
            seg = xc.astype(jnp.int32)
            u = xs - seg.astype(jnp.float32)
            ca = plsc.load_gather(ta_v, [seg])
            cb = plsc.load_gather(tb_v, [seg])
            cc = plsc.load_gather(tc_v, [seg])
            cd = plsc.load_gather(td_v, [seg])
            yref[pl.ds(off, L)] = ca + u * (cb + u * (cc + u * cd))

    def _pair(g, carry):
        for b in range(2):
            off = base + (2 * g + b) * CH
            @pl.when(g > 0)
            def _():
                pltpu.make_async_copy(
                    yb[b], out_hbm.at[pl.ds(off, CH)], sem_out[b]).wait()

            pltpu.make_async_copy(
                x_hbm.at[pl.ds(off, CH)], xb[b], sem_in[b]).wait()
            _compute(xb[b], yb[b])
            pltpu.async_copy(
                yb[b], out_hbm.at[pl.ds(off, CH)], sem_out[b])

            @pl.when(g < NCH // 2 - 1)
            def _():
                pltpu.async_copy(
                    x_hbm.at[pl.ds(off + 2 * CH, CH)], xb[b], sem_in[b])
        return carry

    lax.fori_loop(0, NCH // 2, _pair, 0)

    pltpu.make_async_copy(yb[0], out_hbm.at[pl.ds(base, CH)], sem_out[0]).wait()
    pltpu.make_async_copy(yb[1], out_hbm.at[pl.ds(base, CH)], sem_out[1]).wait()


_spline_sc = pl.kernel(
    _spline_body,
    out_type=jax.ShapeDtypeStruct((N,), jnp.float32),
    mesh=plsc.VectorSubcoreMesh(core_axis_name="c", subcore_axis_name="s"),
    scratch_types=[
        pltpu.VMEM((NUM_SEGMENTS + 1,), jnp.float32),
        pltpu.VMEM((NUM_SEGMENTS + 1,), jnp.float32),
        pltpu.VMEM((NUM_SEGMENTS,), jnp.float32),
        pltpu.VMEM((NUM_SEGMENTS,), jnp.float32),
        pltpu.VMEM((NUM_SEGMENTS,), jnp.float32),
        pltpu.VMEM((NUM_SEGMENTS,), jnp.float32),
        pltpu.VMEM((CH,), jnp.float32),
        pltpu.VMEM((CH,), jnp.float32),
        pltpu.VMEM((CH,), jnp.float32),
        pltpu.VMEM((CH,), jnp.float32),
        pltpu.SemaphoreType.DMA,
        pltpu.SemaphoreType.DMA,
        pltpu.SemaphoreType.DMA,
        pltpu.SemaphoreType.DMA,
    ],
    compiler_params=pltpu.CompilerParams(needs_layout_passes=False),
)


@jax.jit
def kernel(x, control_points, derivatives):
    y = _spline_sc(x, control_points[:, 0], derivatives[:, 0])
    return y[:, None]
